# Initial kernel scaffold; baseline (speedup 1.0000x reference)
import jax
import jax.numpy as jnp
from jax.experimental import pallas as pl


def kernel(indices, enhancement, edition, seal, segment, suit, rank, scalar_properties, debuffed, idx_table, enh_table, edi_table, seal_table, seg_table, suit_table, rank_table):
    raise NotImplementedError("write your pallas kernel here")


# speedup vs baseline: 7.3576x; 7.3576x over previous
import functools
import math

import jax
import jax.numpy as jnp
from jax import lax
from jax.experimental import pallas as pl
from jax.experimental.pallas import tpu as pltpu
from jax.experimental.pallas import tpu_sc as plsc

B, L = 4096, 50
N = B * L
OUT_D = 63
G_D = 48
S_D = 16
NC, NS = 2, 16
NW = NC * NS
ROWS_PER_W = B // NW
R_TILE = 4
E_TILE = R_TILE * L
N_TILES = ROWS_PER_W // R_TILE


def _body(tpad_h, idx_h, enh_h, edi_h, seal_h, seg_h, suit_h, rank_h,
          dbf_h, scal_h, enht_h, edit_h, sealt_h, segt_h, sin_h, cos_h,
          out_h, mask_h,
          out_t, g48, sp48, mask_t, idx_v, enh_v, edi_v, seal_v, seg_v, suit_v,
          rank_v, dbf_v, scal_v, enh_t, edi_t, seal_t, seg_t, sin_t,
          cos_t, rhist, shist, sem):
    wid = lax.axis_index("s") * NC + lax.axis_index("c")
    pltpu.sync_copy(enht_h, enh_t)
    pltpu.sync_copy(edit_h, edi_t)
    pltpu.sync_copy(sealt_h, seal_t)
    pltpu.sync_copy(segt_h, seg_t)
    pltpu.sync_copy(sin_h, sin_t)
    pltpu.sync_copy(cos_h, cos_t)

    iota = lax.iota(jnp.int32, 16)
    ones = jnp.ones((16,), jnp.float32)
    base_w = wid * (ROWS_PER_W * L)

    def col(c):
        return jnp.full((16,), c, jnp.int32)

    def tile_body(t, carry):
        base = base_w + t * E_TILE
        pltpu.sync_copy(idx_h.at[pl.ds(base, E_TILE)], idx_v)
        pltpu.sync_copy(enh_h.at[pl.ds(base, E_TILE)], enh_v)
        pltpu.sync_copy(edi_h.at[pl.ds(base, E_TILE)], edi_v)
        pltpu.sync_copy(seal_h.at[pl.ds(base, E_TILE)], seal_v)
        pltpu.sync_copy(seg_h.at[pl.ds(base, E_TILE)], seg_v)
        pltpu.sync_copy(suit_h.at[pl.ds(base, E_TILE)], suit_v)
        pltpu.sync_copy(rank_h.at[pl.ds(base, E_TILE)], rank_v)
        pltpu.sync_copy(dbf_h.at[pl.ds(base, E_TILE)], dbf_v)
        pltpu.sync_copy(scal_h.at[pl.ds(base * 4, E_TILE * 4)], scal_v)
        pltpu.async_copy(tpad_h.at[idx_v.at[pl.ds(0, 128)]],
                         g48.at[pl.ds(0, 128)], sem).wait()
        pltpu.async_copy(tpad_h.at[idx_v.at[pl.ds(128, E_TILE - 128)]],
                         g48.at[pl.ds(128, E_TILE - 128)], sem).wait()
        sid = lax.axis_index("s")
        pltpu.sync_copy(g48, sp48.at[sid])
        pltpu.sync_copy(sp48.at[sid], out_t.at[:, pl.ds(0, G_D)])

        for r in range(R_TILE):
            e0 = r * L
            rhist[...] = jnp.zeros((16,), jnp.float32)
            shist[...] = jnp.zeros((16,), jnp.float32)
            groups = []
            for g in range(4):
                rem = min(16, L - 16 * g)
                valid = iota < rem
                eidx = jnp.minimum(e0 + 16 * g + iota, e0 + L - 1)
                rv = plsc.load_gather(rank_v, [eidx])
                rv = jnp.where(valid, rv, 15)
                sv = plsc.load_gather(suit_v, [eidx])
                sv = jnp.where(valid, sv, 15)
                groups.append((eidx, valid, rv, sv))
                plsc.addupdate_scatter(rhist, [rv], ones, mask=valid)
                plsc.addupdate_scatter(shist, [sv], ones, mask=valid)
            for eidx, valid, rv, sv in groups:
                same_rank = plsc.load_gather(rhist, [rv])
                same_suit = plsc.load_gather(shist, [sv])
                up = plsc.load_gather(rhist, [jnp.maximum(rv - 1, 0)])
                down = plsc.load_gather(rhist, [jnp.minimum(rv + 1, 15)])
                m0 = rv == 0
                zero = jnp.zeros((16,), jnp.float32)
                f_sr = jnp.where(m0, zero, same_rank * 0.2)
                f_ss = jnp.where(m0, zero, same_suit * 0.2)
                f_up = jnp.where(m0, zero, up)
                f_dn = jnp.where(m0, zero, down)
                sinv = plsc.load_gather(sin_t, [rv])
                cosv = plsc.load_gather(cos_t, [rv])
                plsc.addupdate_scatter(out_t, [eidx, col(37)], cosv, mask=valid)
                plsc.addupdate_scatter(out_t, [eidx, col(38)], sinv, mask=valid)
                plsc.addupdate_scatter(out_t, [eidx, col(39)], f_dn, mask=valid)
                plsc.addupdate_scatter(out_t, [eidx, col(40)], f_up, mask=valid)
                plsc.addupdate_scatter(out_t, [eidx, col(41)], f_ss, mask=valid)
                plsc.addupdate_scatter(out_t, [eidx, col(42)], f_sr, mask=valid)
                plsc.addupdate_scatter(out_t, [eidx, sv], ones, mask=valid)
                plsc.addupdate_scatter(out_t, [eidx, rv + 5], ones, mask=valid)
                iv = plsc.load_gather(idx_v, [eidx])
                mo = jnp.where((iv == 0) & m0,
                               jnp.ones((16,), jnp.int32),
                               jnp.zeros((16,), jnp.int32))
                plsc.store_scatter(mask_t, [eidx], mo, mask=valid)
                for vals_ref, tab_ref, c0 in ((seg_v, seg_t, 47),
                                              (enh_v, enh_t, 51),
                                              (edi_v, edi_t, 55),
                                              (seal_v, seal_t, 59)):
                    fv = plsc.load_gather(vals_ref, [eidx])
                    for c in range(4):
                        plsc.store_scatter(out_t, [eidx, col(c0 + c)],
                                           plsc.load_gather(tab_ref, [fv, col(c)]),
                                           mask=valid)
                dbf = plsc.load_gather(dbf_v, [eidx])
                ndb = 1.0 - dbf.astype(jnp.float32)
                for c, inv in enumerate((0.1, 0.01, 0.01, 0.1)):
                    s = plsc.load_gather(scal_v, [eidx * 4 + c])
                    plsc.store_scatter(out_t, [eidx, col(43 + c)],
                                       s * ndb * inv, mask=valid)
        pltpu.sync_copy(out_t, out_h.at[pl.ds(base, E_TILE)])
        pltpu.sync_copy(mask_t, mask_h.at[pl.ds(base, E_TILE)])
        return carry

    lax.fori_loop(0, N_TILES, tile_body, 0)


@jax.jit
def _run(tpad, idx_f, enh_f, edi_f, seal_f, seg_f, suit_f, rank_f, dbf_f,
         scal_f, enh_table, edi_table, seal_table, seg_table, sin_tab,
         cos_tab):
    mesh = plsc.VectorSubcoreMesh(core_axis_name="c", subcore_axis_name="s")
    f = pl.kernel(
        _body,
        out_type=(jax.ShapeDtypeStruct((N, OUT_D), jnp.float32),
                  jax.ShapeDtypeStruct((N,), jnp.int32)),
        mesh=mesh,
        compiler_params=pltpu.CompilerParams(needs_layout_passes=False,
                                             use_tc_tiling_on_sc=False),
        scratch_types=[
            pltpu.VMEM((E_TILE, OUT_D), jnp.float32),
            pltpu.VMEM((E_TILE, G_D), jnp.float32),
            pltpu.VMEM_SHARED((NS, E_TILE, G_D), jnp.float32),
            pltpu.VMEM((E_TILE,), jnp.int32),
            pltpu.VMEM((E_TILE,), jnp.int32),
            pltpu.VMEM((E_TILE,), jnp.int32),
            pltpu.VMEM((E_TILE,), jnp.int32),
            pltpu.VMEM((E_TILE,), jnp.int32),
            pltpu.VMEM((E_TILE,), jnp.int32),
            pltpu.VMEM((E_TILE,), jnp.int32),
            pltpu.VMEM((E_TILE,), jnp.int32),
            pltpu.VMEM((E_TILE,), jnp.int32),
            pltpu.VMEM((E_TILE * 4,), jnp.float32),
            pltpu.VMEM((16, 4), jnp.float32),
            pltpu.VMEM((8, 4), jnp.float32),
            pltpu.VMEM((8, 4), jnp.float32),
            pltpu.VMEM((16, 4), jnp.float32),
            pltpu.VMEM((16,), jnp.float32),
            pltpu.VMEM((16,), jnp.float32),
            pltpu.VMEM((16,), jnp.float32),
            pltpu.VMEM((16,), jnp.float32),
            pltpu.SemaphoreType.DMA,
        ],
    )
    return f(tpad, idx_f, enh_f, edi_f, seal_f, seg_f, suit_f, rank_f,
             dbf_f, scal_f, enh_table, edi_table, seal_table, seg_table,
             sin_tab, cos_tab)


def kernel(indices, enhancement, edition, seal, segment, suit, rank,
           scalar_properties, debuffed,
           idx_table, enh_table, edi_table, seal_table, seg_table,
           suit_table, rank_table):
    tpad = jnp.pad(idx_table, ((0, 0), (0, G_D - idx_table.shape[1])))
    ang = jnp.arange(16, dtype=jnp.float32) * (2.0 * math.pi / 13.0)
    sin_tab = jnp.sin(ang)
    cos_tab = jnp.cos(ang)
    emb_f, mask_f = _run(
        tpad,
        indices.reshape(-1), enhancement.reshape(-1), edition.reshape(-1),
        seal.reshape(-1), segment.reshape(-1), suit.reshape(-1),
        rank.reshape(-1), debuffed.reshape(-1),
        scalar_properties.reshape(-1),
        enh_table, edi_table, seal_table, seg_table, sin_tab, cos_tab)
    return emb_f.reshape(B, L, OUT_D), mask_f.reshape(B, L) != 0


# speedup vs baseline: 10.7327x; 1.4587x over previous
import functools
import math

import jax
import jax.numpy as jnp
from jax import lax
from jax.experimental import pallas as pl
from jax.experimental.pallas import tpu as pltpu
from jax.experimental.pallas import tpu_sc as plsc

B, L = 4096, 50
N = B * L
OUT_D = 63
G_D = 48
S_D = 16
NC, NS = 2, 16
NW = NC * NS
ROWS_PER_W = B // NW
R_TILE = 8
E_TILE = R_TILE * L
N_TILES = ROWS_PER_W // R_TILE


def _body(tpad_h, idx_h, enh_h, edi_h, seal_h, seg_h, suit_h, rank_h,
          dbf_h, scal_h, enht_h, edit_h, sealt_h, segt_h, sin_h, cos_h,
          out_h, mask_h,
          out_t, g48, sp48, mask_t, idx_v, enh_v, edi_v, seal_v, seg_v, suit_v,
          rank_v, dbf_v, scal_v, enh_t, edi_t, seal_t, seg_t, sin_t,
          cos_t, rhist, shist, sem):
    wid = lax.axis_index("s") * NC + lax.axis_index("c")
    pltpu.sync_copy(enht_h, enh_t)
    pltpu.sync_copy(edit_h, edi_t)
    pltpu.sync_copy(sealt_h, seal_t)
    pltpu.sync_copy(segt_h, seg_t)
    pltpu.sync_copy(sin_h, sin_t)
    pltpu.sync_copy(cos_h, cos_t)

    iota = lax.iota(jnp.int32, 16)
    ones = jnp.ones((16,), jnp.float32)
    base_w = wid * (ROWS_PER_W * L)

    def col(c):
        return jnp.full((16,), c, jnp.int32)

    sid = lax.axis_index("s")

    def tile_body(t, carry):
        base = base_w + t * E_TILE
        row_base = wid * ROWS_PER_W + t * R_TILE
        d_idx = pltpu.async_copy(idx_h.at[pl.ds(base, E_TILE)], idx_v, sem)
        others = [
            pltpu.async_copy(enh_h.at[pl.ds(base, E_TILE)], enh_v, sem),
            pltpu.async_copy(edi_h.at[pl.ds(base, E_TILE)], edi_v, sem),
            pltpu.async_copy(seal_h.at[pl.ds(base, E_TILE)], seal_v, sem),
            pltpu.async_copy(seg_h.at[pl.ds(base, E_TILE)], seg_v, sem),
            pltpu.async_copy(suit_h.at[pl.ds(base, E_TILE)], suit_v, sem),
            pltpu.async_copy(rank_h.at[pl.ds(base, E_TILE)], rank_v, sem),
            pltpu.async_copy(dbf_h.at[pl.ds(base, E_TILE)], dbf_v, sem),
            pltpu.async_copy(scal_h.at[pl.ds(base * 4, E_TILE * 4)], scal_v,
                             sem),
        ]
        d_idx.wait()
        gathers = []
        for lo in range(0, E_TILE, 128):
            n = min(128, E_TILE - lo)
            gathers.append(pltpu.async_copy(
                tpad_h.at[idx_v.at[pl.ds(lo, n)]], g48.at[pl.ds(lo, n)], sem))
        for d in others + gathers:
            d.wait()
        pltpu.sync_copy(g48, sp48.at[sid])
        pltpu.sync_copy(sp48.at[sid], out_t.at[:, pl.ds(0, G_D)])

        for r in range(R_TILE):
            e0 = r * L
            rhist[...] = jnp.zeros((16,), jnp.float32)
            shist[...] = jnp.zeros((16,), jnp.float32)
            groups = []
            for g in range(4):
                rem = min(16, L - 16 * g)
                valid = iota < rem
                eidx = jnp.minimum(e0 + 16 * g + iota, e0 + L - 1)
                rv = plsc.load_gather(rank_v, [eidx])
                rv = jnp.where(valid, rv, 15)
                sv = plsc.load_gather(suit_v, [eidx])
                sv = jnp.where(valid, sv, 15)
                groups.append((eidx, valid, rv, sv))
                plsc.addupdate_scatter(rhist, [rv], ones, mask=valid)
                plsc.addupdate_scatter(shist, [sv], ones, mask=valid)
            for eidx, valid, rv, sv in groups:
                same_rank = plsc.load_gather(rhist, [rv])
                same_suit = plsc.load_gather(shist, [sv])
                up = plsc.load_gather(rhist, [jnp.maximum(rv - 1, 0)])
                down = plsc.load_gather(rhist, [jnp.minimum(rv + 1, 15)])
                m0 = rv == 0
                zero = jnp.zeros((16,), jnp.float32)
                f_sr = jnp.where(m0, zero, same_rank * 0.2)
                f_ss = jnp.where(m0, zero, same_suit * 0.2)
                f_up = jnp.where(m0, zero, up)
                f_dn = jnp.where(m0, zero, down)
                sinv = plsc.load_gather(sin_t, [rv])
                cosv = plsc.load_gather(cos_t, [rv])
                plsc.addupdate_scatter(out_t, [eidx, col(37)], cosv, mask=valid)
                plsc.addupdate_scatter(out_t, [eidx, col(38)], sinv, mask=valid)
                plsc.addupdate_scatter(out_t, [eidx, col(39)], f_dn, mask=valid)
                plsc.addupdate_scatter(out_t, [eidx, col(40)], f_up, mask=valid)
                plsc.addupdate_scatter(out_t, [eidx, col(41)], f_ss, mask=valid)
                plsc.addupdate_scatter(out_t, [eidx, col(42)], f_sr, mask=valid)
                plsc.addupdate_scatter(out_t, [eidx, sv], ones, mask=valid)
                plsc.addupdate_scatter(out_t, [eidx, rv + 5], ones, mask=valid)
                iv = plsc.load_gather(idx_v, [eidx])
                mo = jnp.where((iv == 0) & m0,
                               jnp.ones((16,), jnp.int32),
                               jnp.zeros((16,), jnp.int32))
                plsc.store_scatter(mask_t, [eidx], mo, mask=valid)
                for vals_ref, tab_ref, c0 in ((seg_v, seg_t, 47),
                                              (enh_v, enh_t, 51),
                                              (edi_v, edi_t, 55),
                                              (seal_v, seal_t, 59)):
                    fv = plsc.load_gather(vals_ref, [eidx])
                    for c in range(4):
                        plsc.store_scatter(out_t, [eidx, col(c0 + c)],
                                           plsc.load_gather(tab_ref, [fv, col(c)]),
                                           mask=valid)
                dbf = plsc.load_gather(dbf_v, [eidx])
                ndb = 1.0 - dbf.astype(jnp.float32)
                for c, inv in enumerate((0.1, 0.01, 0.01, 0.1)):
                    s = plsc.load_gather(scal_v, [eidx * 4 + c])
                    plsc.store_scatter(out_t, [eidx, col(43 + c)],
                                       s * ndb * inv, mask=valid)
        outs = [pltpu.async_copy(out_t.at[pl.ds(L * r, L)],
                                 out_h.at[row_base + r], sem)
                for r in range(R_TILE)]
        outs.append(pltpu.async_copy(mask_t, mask_h.at[pl.ds(base, E_TILE)],
                                     sem))
        for d in outs:
            d.wait()
        return carry

    lax.fori_loop(0, N_TILES, tile_body, 0)


@jax.jit
def _run(tpad, idx_f, enh_f, edi_f, seal_f, seg_f, suit_f, rank_f, dbf_f,
         scal_f, enh_table, edi_table, seal_table, seg_table, sin_tab,
         cos_tab):
    mesh = plsc.VectorSubcoreMesh(core_axis_name="c", subcore_axis_name="s")
    f = pl.kernel(
        _body,
        out_type=(jax.ShapeDtypeStruct((B, L, OUT_D), jnp.float32),
                  jax.ShapeDtypeStruct((N,), jnp.int32)),
        mesh=mesh,
        compiler_params=pltpu.CompilerParams(needs_layout_passes=False,
                                             use_tc_tiling_on_sc=False),
        scratch_types=[
            pltpu.VMEM((E_TILE, OUT_D), jnp.float32),
            pltpu.VMEM((E_TILE, G_D), jnp.float32),
            pltpu.VMEM_SHARED((NS, E_TILE, G_D), jnp.float32),
            pltpu.VMEM((E_TILE,), jnp.int32),
            pltpu.VMEM((E_TILE,), jnp.int32),
            pltpu.VMEM((E_TILE,), jnp.int32),
            pltpu.VMEM((E_TILE,), jnp.int32),
            pltpu.VMEM((E_TILE,), jnp.int32),
            pltpu.VMEM((E_TILE,), jnp.int32),
            pltpu.VMEM((E_TILE,), jnp.int32),
            pltpu.VMEM((E_TILE,), jnp.int32),
            pltpu.VMEM((E_TILE,), jnp.int32),
            pltpu.VMEM((E_TILE * 4,), jnp.float32),
            pltpu.VMEM((16, 4), jnp.float32),
            pltpu.VMEM((8, 4), jnp.float32),
            pltpu.VMEM((8, 4), jnp.float32),
            pltpu.VMEM((16, 4), jnp.float32),
            pltpu.VMEM((16,), jnp.float32),
            pltpu.VMEM((16,), jnp.float32),
            pltpu.VMEM((16,), jnp.float32),
            pltpu.VMEM((16,), jnp.float32),
            pltpu.SemaphoreType.DMA,
        ],
    )
    return f(tpad, idx_f, enh_f, edi_f, seal_f, seg_f, suit_f, rank_f,
             dbf_f, scal_f, enh_table, edi_table, seal_table, seg_table,
             sin_tab, cos_tab)


def kernel(indices, enhancement, edition, seal, segment, suit, rank,
           scalar_properties, debuffed,
           idx_table, enh_table, edi_table, seal_table, seg_table,
           suit_table, rank_table):
    tpad = jnp.pad(idx_table, ((0, 0), (0, G_D - idx_table.shape[1])))
    ang = jnp.arange(16, dtype=jnp.float32) * (2.0 * math.pi / 13.0)
    sin_tab = jnp.sin(ang)
    cos_tab = jnp.cos(ang)
    emb_f, mask_f = _run(
        tpad,
        indices.reshape(-1), enhancement.reshape(-1), edition.reshape(-1),
        seal.reshape(-1), segment.reshape(-1), suit.reshape(-1),
        rank.reshape(-1), debuffed.reshape(-1),
        scalar_properties.reshape(-1),
        enh_table, edi_table, seal_table, seg_table, sin_tab, cos_tab)
    return emb_f, mask_f.reshape(B, L) != 0
